# baseline (speedup 1.0000x reference)
import functools

import jax
import jax.numpy as jnp
from jax import lax
from jax.experimental import pallas as pl
from jax.experimental.pallas import tpu as pltpu, tpu_sc as plsc

N = 10000
E = 320000
D = 128
L = 3

NC = 2
NS = 16
NW = NC * NS
EPW = E // NW
CH = 80
NCHUNK = EPW // CH
BCH = 5
NBLK = NCHUNK // BCH
ZR = 40
NZCH = N // ZR


def _ee_body(a_ref, w_ref, b_ref, o_ref):
    o_ref[...] = (
        jnp.dot(a_ref[...], w_ref[...], preferred_element_type=jnp.float32)
        + b_ref[...]
    )


def _edge_embed(attr8, w8, b):
    BE = 8000
    return pl.pallas_call(
        _ee_body,
        grid=(E // BE,),
        in_specs=[
            pl.BlockSpec((BE, 8), lambda i: (i, 0)),
            pl.BlockSpec((8, D), lambda i: (0, 0)),
            pl.BlockSpec((1, D), lambda i: (0, 0)),
        ],
        out_specs=pl.BlockSpec((BE, D), lambda i: (i, 0)),
        out_shape=jax.ShapeDtypeStruct((E, D), jnp.float32),
    )(attr8, w8, b)


def _sc_agg_body(h_hbm, ee_hbm, row3_hbm, col3_hbm, out_hbm,
                 rib0_v, rib1_v, cib0_v, cib1_v, rows0_v, rows1_v,
                 ee0_v, ee1_v, zb_v, agg_sh,
                 gsem0, gsem1, esem0, esem1, bsem0, bsem1, ssem0, ssem1,
                 *, uniform_h):
    c = lax.axis_index("c")
    s = lax.axis_index("s")
    wid = s * NC + c
    blk0 = wid * NBLK
    chk0 = wid * NCHUNK
    rib = (rib0_v, rib1_v)
    cib = (cib0_v, cib1_v)
    rows = (rows0_v, rows1_v)
    ees = (ee0_v, ee1_v)
    gsem = (gsem0, gsem1)
    esem = (esem0, esem1)
    bsem = (bsem0, bsem1)
    ssem = (ssem0, ssem1)

    def _zero_body(i, _):
        for j in range(8):
            zb_v[i, pl.ds(j * 16, 16)] = jnp.zeros((16,), jnp.float32)
        return 0

    lax.fori_loop(0, ZR, _zero_body, 0)
    for k in range((NZCH + NS - 1) // NS):
        idx = s + k * NS

        @pl.when(idx < NZCH)
        def _():
            pltpu.sync_copy(zb_v, agg_sh.at[pl.ds(idx * ZR, ZR)])

    plsc.subcore_barrier()

    def _bcopy(blk, bslot, sem):
        if not uniform_h:
            pltpu.async_copy(row3_hbm.at[blk], rib[bslot], sem)
        pltpu.async_copy(col3_hbm.at[blk], cib[bslot], sem)

    def _bwait(bslot, sem):
        if not uniform_h:
            pltpu.make_async_copy(row3_hbm.at[0], rib[bslot], sem).wait()
        pltpu.make_async_copy(col3_hbm.at[0], cib[bslot], sem).wait()

    def _issue(echunk, slot, bslot, j):
        if not uniform_h:
            pltpu.async_copy(h_hbm.at[rib[bslot].at[j]], rows[slot],
                             gsem[slot])
        pltpu.async_copy(ee_hbm.at[pl.ds(echunk * CH, CH)], ees[slot],
                         esem[slot])

    def _wait(slot):
        if not uniform_h:
            pltpu.make_async_copy(h_hbm.at[rib[0].at[0]], rows[slot],
                                  gsem[slot]).wait()
        pltpu.make_async_copy(ee_hbm.at[pl.ds(0, CH)], ees[slot],
                              esem[slot]).wait()

    if uniform_h:
        pltpu.sync_copy(h_hbm.at[pl.ds(0, 8)], zb_v.at[pl.ds(0, 8)])
        h0r = [zb_v[0, pl.ds(jj * 16, 16)] for jj in range(8)]
    else:
        h0r = None

    def _compute_scatter(slot, bslot, j):
        rv = rows[slot]
        ev = ees[slot]

        if uniform_h:
            @plsc.parallel_loop(0, CH, 1, unroll=2)
            def _edge_body(e):
                for jj in range(8):
                    sl = pl.ds(jj * 16, 16)
                    ev[e, sl] = jnp.maximum(ev[e, sl] + h0r[jj], 0.0)
        else:
            @plsc.parallel_loop(0, CH, 1, unroll=2)
            def _edge_body(e):
                for jj in range(8):
                    sl = pl.ds(jj * 16, 16)
                    rv[e, sl] = jnp.maximum(rv[e, sl] + ev[e, sl], 0.0)
        src = ees[slot] if uniform_h else rv
        pltpu.async_copy(src, agg_sh.at[cib[bslot].at[j]], ssem[slot],
                         add=True)

    def _swait(slot):
        pltpu.make_async_copy(rows[slot], agg_sh.at[cib[0].at[0]],
                              ssem[slot]).wait()

    def _steps(base_chunk, blk_a):
        for m in range(2 * BCH):
            slot = m % 2
            nslot = 1 - slot
            bslot = 0 if m < BCH else 1
            j = m % BCH
            if m == BCH - 1:
                _bwait(1, bsem1)
            if m == 2 * BCH - 1:
                _bwait(0, bsem0)
                _swait(nslot)
                _issue(base_chunk + 2 * BCH, nslot, 0, 0)
            else:
                nc = m + 1
                _swait(nslot)
                _issue(base_chunk + nc, nslot, 0 if nc < BCH else 1, nc % BCH)
            if m == 0:
                _bcopy(blk_a + 1, 1, bsem1)
            if m == BCH:
                _bcopy(blk_a + 2, 0, bsem0)
            _wait(slot)
            _compute_scatter(slot, bslot, j)

    pltpu.sync_copy(row3_hbm.at[blk0], rib0_v)
    pltpu.sync_copy(col3_hbm.at[blk0], cib0_v)

    def _zr_body(e, _):
        for jj in range(8):
            rows1_v[e, pl.ds(jj * 16, 16)] = jnp.zeros((16,), jnp.float32)
        return 0

    lax.fori_loop(0, CH, _zr_body, 0)
    pltpu.async_copy(rows1_v, agg_sh.at[cib0_v.at[0]], ssem1, add=True)
    _issue(chk0, 0, 0, 0)

    def _pair_body(it, _):
        _steps(chk0 + 2 * BCH * it, blk0 + 2 * it)
        return 0

    lax.fori_loop(0, NBLK // 2, _pair_body, 0)

    tbase = chk0 + (NBLK - 1) * BCH
    for j in range(BCH):
        slot = j % 2
        if j < BCH - 1:
            _swait(1 - slot)
            _issue(tbase + j + 1, 1 - slot, 0, j + 1)
        _wait(slot)
        _compute_scatter(slot, 0, j)
    _swait(0)
    _swait(1)
    plsc.subcore_barrier()

    for k in range((NZCH + NS - 1) // NS):
        idx = s + k * NS

        @pl.when(idx < NZCH)
        def _():
            r0 = idx * ZR
            pltpu.sync_copy(agg_sh.at[pl.ds(r0, ZR)], zb_v)
            pltpu.sync_copy(zb_v, out_hbm.at[c].at[pl.ds(r0, ZR)])


@functools.lru_cache(maxsize=None)
def _make_sc_agg(uniform_h=False):
    return pl.kernel(
        functools.partial(_sc_agg_body, uniform_h=uniform_h),
        out_type=jax.ShapeDtypeStruct((NC, N, D), jnp.float32),
        mesh=plsc.VectorSubcoreMesh(
            core_axis_name="c", subcore_axis_name="s",
            num_cores=NC, num_subcores=NS,
        ),
        scratch_types=[
            pltpu.VMEM((BCH, CH), jnp.int32),
            pltpu.VMEM((BCH, CH), jnp.int32),
            pltpu.VMEM((BCH, CH), jnp.int32),
            pltpu.VMEM((BCH, CH), jnp.int32),
            pltpu.VMEM((CH, D), jnp.float32),
            pltpu.VMEM((CH, D), jnp.float32),
            pltpu.VMEM((CH, D), jnp.float32),
            pltpu.VMEM((CH, D), jnp.float32),
            pltpu.VMEM((ZR, D), jnp.float32),
            pltpu.VMEM_SHARED((N, D), jnp.float32),
            pltpu.SemaphoreType.DMA,
            pltpu.SemaphoreType.DMA,
            pltpu.SemaphoreType.DMA,
            pltpu.SemaphoreType.DMA,
            pltpu.SemaphoreType.DMA,
            pltpu.SemaphoreType.DMA,
            pltpu.SemaphoreType.DMA,
            pltpu.SemaphoreType.DMA,
        ],
    )


def _mlp_body(eps_ref, h_ref, agg_ref, w1_ref, b1_ref, gm_ref, bm_ref,
              w2_ref, b2_ref, g_ref, be_ref, o_ref, *, final):
    z = (1.0 + eps_ref[0]) * h_ref[...] + agg_ref[0] + agg_ref[1]
    y = jnp.dot(z, w1_ref[...], preferred_element_type=jnp.float32) + b1_ref[...]
    mu = jnp.mean(y, axis=0, keepdims=True)
    var = jnp.mean((y - mu) ** 2, axis=0, keepdims=True)
    y = (y - mu) / jnp.sqrt(var + 1e-5) * gm_ref[...] + bm_ref[...]
    y = jnp.maximum(y, 0.0)
    y2 = jnp.dot(y, w2_ref[...], preferred_element_type=jnp.float32) + b2_ref[...]
    mu2 = jnp.mean(y2, axis=0, keepdims=True)
    var2 = jnp.mean((y2 - mu2) ** 2, axis=0, keepdims=True)
    y2 = (y2 - mu2) / jnp.sqrt(var2 + 1e-5) * g_ref[...] + be_ref[...]
    if not final:
        y2 = jnp.maximum(y2, 0.0)
    o_ref[...] = y2


def _mlp(eps, h, agg, w1, b1, gm, bm, w2, b2, g, be, final):
    body = functools.partial(_mlp_body, final=final)
    return pl.pallas_call(
        body,
        in_specs=[
            pl.BlockSpec(memory_space=pltpu.SMEM),
            pl.BlockSpec((N, D), lambda: (0, 0)),
            pl.BlockSpec((NC, N, D), lambda: (0, 0, 0)),
            pl.BlockSpec((D, 2 * D), lambda: (0, 0)),
            pl.BlockSpec((1, 2 * D), lambda: (0, 0)),
            pl.BlockSpec((1, 2 * D), lambda: (0, 0)),
            pl.BlockSpec((1, 2 * D), lambda: (0, 0)),
            pl.BlockSpec((2 * D, D), lambda: (0, 0)),
            pl.BlockSpec((1, D), lambda: (0, 0)),
            pl.BlockSpec((1, D), lambda: (0, 0)),
            pl.BlockSpec((1, D), lambda: (0, 0)),
        ],
        out_specs=pl.BlockSpec((N, D), lambda: (0, 0)),
        out_shape=jax.ShapeDtypeStruct((N, D), jnp.float32),
    )(eps, h, agg, w1, b1, gm, bm, w2, b2, g, be)


def kernel(x, edge_index, edge_attr, batch, node_enc_W, edge_W, edge_b,
           W1, b1, g_mid, be_mid, W2, b2, eps_arr, gamma, beta):
    h = jnp.take(node_enc_W, x, axis=0)
    row3 = edge_index[0].reshape(NW * NBLK, BCH, CH)
    col3 = edge_index[1].reshape(NW * NBLK, BCH, CH)
    attr8 = jnp.pad(edge_attr, ((0, 0), (0, 1)))

    for l in range(L):
        w8 = jnp.pad(edge_W[l], ((0, 1), (0, 0)))
        ee = _edge_embed(attr8, w8, edge_b[l].reshape(1, D))
        agg = _make_sc_agg(uniform_h=(l == 0))(h, ee, row3, col3)
        h = _mlp(
            eps_arr[l].reshape(1), h, agg,
            W1[l], b1[l].reshape(1, 2 * D),
            g_mid[l].reshape(1, 2 * D), be_mid[l].reshape(1, 2 * D),
            W2[l], b2[l].reshape(1, D),
            gamma[l].reshape(1, D), beta[l].reshape(1, D),
            final=(l == L - 1),
        )
    return h


# speedup vs baseline: 1.0122x; 1.0122x over previous
import functools

import jax
import jax.numpy as jnp
from jax import lax
from jax.experimental import pallas as pl
from jax.experimental.pallas import tpu as pltpu, tpu_sc as plsc

N = 10000
E = 320000
D = 128
L = 3

NC = 2
NS = 16
NW = NC * NS
EPW = E // NW
CH = 80
NCHUNK = EPW // CH
BCH = 5
NBLK = NCHUNK // BCH
ZR = 40
NZCH = N // ZR


def _ee_body(a_ref, w_ref, b_ref, o_ref):
    o_ref[...] = (
        jnp.dot(a_ref[...], w_ref[...], preferred_element_type=jnp.float32)
        + b_ref[...]
    )


def _edge_embed(attr8, w8, b):
    BE = 8000
    return pl.pallas_call(
        _ee_body,
        grid=(E // BE,),
        in_specs=[
            pl.BlockSpec((BE, 8), lambda i: (i, 0)),
            pl.BlockSpec((8, D), lambda i: (0, 0)),
            pl.BlockSpec((1, D), lambda i: (0, 0)),
        ],
        out_specs=pl.BlockSpec((BE, D), lambda i: (i, 0)),
        out_shape=jax.ShapeDtypeStruct((E, D), jnp.float32),
    )(attr8, w8, b)


def _sc_agg_body(h_hbm, ee_hbm, row3_hbm, col3_hbm, out_hbm,
                 rib0_v, rib1_v, cib0_v, cib1_v, rows0_v, rows1_v,
                 ee0_v, ee1_v, zb_v, agg_sh,
                 gsem0, gsem1, esem0, esem1, bsem0, bsem1, ssem0, ssem1,
                 *, uniform_h):
    c = lax.axis_index("c")
    s = lax.axis_index("s")
    wid = s * NC + c
    blk0 = wid * NBLK
    chk0 = wid * NCHUNK
    rib = (rib0_v, rib1_v)
    cib = (cib0_v, cib1_v)
    rows = (rows0_v, rows1_v)
    ees = (ee0_v, ee1_v)
    gsem = (gsem0, gsem1)
    esem = (esem0, esem1)
    bsem = (bsem0, bsem1)
    ssem = (ssem0, ssem1)

    def _zero_body(i, _):
        for j in range(8):
            zb_v[i, pl.ds(j * 16, 16)] = jnp.zeros((16,), jnp.float32)
        return 0

    lax.fori_loop(0, ZR, _zero_body, 0)
    for k in range((NZCH + NS - 1) // NS):
        idx = s + k * NS

        @pl.when(idx < NZCH)
        def _():
            pltpu.sync_copy(zb_v, agg_sh.at[pl.ds(idx * ZR, ZR)])

    plsc.subcore_barrier()

    def _bcopy(blk, bslot, sem):
        if not uniform_h:
            pltpu.async_copy(row3_hbm.at[blk], rib[bslot], sem)
        pltpu.async_copy(col3_hbm.at[blk], cib[bslot], sem)

    def _bwait(bslot, sem):
        if not uniform_h:
            pltpu.make_async_copy(row3_hbm.at[0], rib[bslot], sem).wait()
        pltpu.make_async_copy(col3_hbm.at[0], cib[bslot], sem).wait()

    def _issue(echunk, slot, bslot, j):
        if not uniform_h:
            pltpu.async_copy(h_hbm.at[rib[bslot].at[j]], rows[slot],
                             gsem[slot])
        pltpu.async_copy(ee_hbm.at[pl.ds(echunk * CH, CH)], ees[slot],
                         esem[slot])

    def _wait(slot):
        if not uniform_h:
            pltpu.make_async_copy(h_hbm.at[rib[0].at[0]], rows[slot],
                                  gsem[slot]).wait()
        pltpu.make_async_copy(ee_hbm.at[pl.ds(0, CH)], ees[slot],
                              esem[slot]).wait()

    if uniform_h:
        pltpu.sync_copy(h_hbm.at[pl.ds(0, 8)], zb_v.at[pl.ds(0, 8)])
        h0r = [zb_v[0, pl.ds(jj * 16, 16)] for jj in range(8)]
    else:
        h0r = None

    def _compute_scatter(slot, bslot, j):
        rv = rows[slot]
        ev = ees[slot]

        if uniform_h:
            def _edge_body(e, _):
                for jj in range(8):
                    sl = pl.ds(jj * 16, 16)
                    ev[e, sl] = jnp.maximum(ev[e, sl] + h0r[jj], 0.0)
                return 0
        else:
            def _edge_body(e, _):
                for jj in range(8):
                    sl = pl.ds(jj * 16, 16)
                    rv[e, sl] = jnp.maximum(rv[e, sl] + ev[e, sl], 0.0)
                return 0

        lax.fori_loop(0, CH, _edge_body, 0)
        src = ees[slot] if uniform_h else rv
        pltpu.async_copy(src, agg_sh.at[cib[bslot].at[j]], ssem[slot],
                         add=True)

    def _swait(slot):
        pltpu.make_async_copy(rows[slot], agg_sh.at[cib[0].at[0]],
                              ssem[slot]).wait()

    def _steps(base_chunk, blk_a):
        for m in range(2 * BCH):
            slot = m % 2
            nslot = 1 - slot
            bslot = 0 if m < BCH else 1
            j = m % BCH
            if m == BCH - 1:
                _bwait(1, bsem1)
            if m == 2 * BCH - 1:
                _bwait(0, bsem0)
                _swait(nslot)
                _issue(base_chunk + 2 * BCH, nslot, 0, 0)
            else:
                nc = m + 1
                _swait(nslot)
                _issue(base_chunk + nc, nslot, 0 if nc < BCH else 1, nc % BCH)
            if m == 0:
                _bcopy(blk_a + 1, 1, bsem1)
            if m == BCH:
                _bcopy(blk_a + 2, 0, bsem0)
            _wait(slot)
            _compute_scatter(slot, bslot, j)

    pltpu.sync_copy(row3_hbm.at[blk0], rib0_v)
    pltpu.sync_copy(col3_hbm.at[blk0], cib0_v)

    def _zr_body(e, _):
        for jj in range(8):
            rows1_v[e, pl.ds(jj * 16, 16)] = jnp.zeros((16,), jnp.float32)
        return 0

    lax.fori_loop(0, CH, _zr_body, 0)
    pltpu.async_copy(rows1_v, agg_sh.at[cib0_v.at[0]], ssem1, add=True)
    _issue(chk0, 0, 0, 0)

    def _pair_body(it, _):
        _steps(chk0 + 2 * BCH * it, blk0 + 2 * it)
        return 0

    lax.fori_loop(0, NBLK // 2, _pair_body, 0)

    tbase = chk0 + (NBLK - 1) * BCH
    for j in range(BCH):
        slot = j % 2
        if j < BCH - 1:
            _swait(1 - slot)
            _issue(tbase + j + 1, 1 - slot, 0, j + 1)
        _wait(slot)
        _compute_scatter(slot, 0, j)
    _swait(0)
    _swait(1)
    plsc.subcore_barrier()

    for k in range((NZCH + NS - 1) // NS):
        idx = s + k * NS

        @pl.when(idx < NZCH)
        def _():
            r0 = idx * ZR
            pltpu.sync_copy(agg_sh.at[pl.ds(r0, ZR)], zb_v)
            pltpu.sync_copy(zb_v, out_hbm.at[c].at[pl.ds(r0, ZR)])


@functools.lru_cache(maxsize=None)
def _make_sc_agg(uniform_h=False):
    return pl.kernel(
        functools.partial(_sc_agg_body, uniform_h=uniform_h),
        out_type=jax.ShapeDtypeStruct((NC, N, D), jnp.float32),
        mesh=plsc.VectorSubcoreMesh(
            core_axis_name="c", subcore_axis_name="s",
            num_cores=NC, num_subcores=NS,
        ),
        scratch_types=[
            pltpu.VMEM((BCH, CH), jnp.int32),
            pltpu.VMEM((BCH, CH), jnp.int32),
            pltpu.VMEM((BCH, CH), jnp.int32),
            pltpu.VMEM((BCH, CH), jnp.int32),
            pltpu.VMEM((CH, D), jnp.float32),
            pltpu.VMEM((CH, D), jnp.float32),
            pltpu.VMEM((CH, D), jnp.float32),
            pltpu.VMEM((CH, D), jnp.float32),
            pltpu.VMEM((ZR, D), jnp.float32),
            pltpu.VMEM_SHARED((N, D), jnp.float32),
            pltpu.SemaphoreType.DMA,
            pltpu.SemaphoreType.DMA,
            pltpu.SemaphoreType.DMA,
            pltpu.SemaphoreType.DMA,
            pltpu.SemaphoreType.DMA,
            pltpu.SemaphoreType.DMA,
            pltpu.SemaphoreType.DMA,
            pltpu.SemaphoreType.DMA,
        ],
    )


def _mlp_body(eps_ref, h_ref, agg_ref, w1_ref, b1_ref, gm_ref, bm_ref,
              w2_ref, b2_ref, g_ref, be_ref, o_ref, *, final):
    z = (1.0 + eps_ref[0]) * h_ref[...] + agg_ref[0] + agg_ref[1]
    y = jnp.dot(z, w1_ref[...], preferred_element_type=jnp.float32) + b1_ref[...]
    mu = jnp.mean(y, axis=0, keepdims=True)
    var = jnp.mean((y - mu) ** 2, axis=0, keepdims=True)
    y = (y - mu) / jnp.sqrt(var + 1e-5) * gm_ref[...] + bm_ref[...]
    y = jnp.maximum(y, 0.0)
    y2 = jnp.dot(y, w2_ref[...], preferred_element_type=jnp.float32) + b2_ref[...]
    mu2 = jnp.mean(y2, axis=0, keepdims=True)
    var2 = jnp.mean((y2 - mu2) ** 2, axis=0, keepdims=True)
    y2 = (y2 - mu2) / jnp.sqrt(var2 + 1e-5) * g_ref[...] + be_ref[...]
    if not final:
        y2 = jnp.maximum(y2, 0.0)
    o_ref[...] = y2


def _mlp(eps, h, agg, w1, b1, gm, bm, w2, b2, g, be, final):
    body = functools.partial(_mlp_body, final=final)
    return pl.pallas_call(
        body,
        in_specs=[
            pl.BlockSpec(memory_space=pltpu.SMEM),
            pl.BlockSpec((N, D), lambda: (0, 0)),
            pl.BlockSpec((NC, N, D), lambda: (0, 0, 0)),
            pl.BlockSpec((D, 2 * D), lambda: (0, 0)),
            pl.BlockSpec((1, 2 * D), lambda: (0, 0)),
            pl.BlockSpec((1, 2 * D), lambda: (0, 0)),
            pl.BlockSpec((1, 2 * D), lambda: (0, 0)),
            pl.BlockSpec((2 * D, D), lambda: (0, 0)),
            pl.BlockSpec((1, D), lambda: (0, 0)),
            pl.BlockSpec((1, D), lambda: (0, 0)),
            pl.BlockSpec((1, D), lambda: (0, 0)),
        ],
        out_specs=pl.BlockSpec((N, D), lambda: (0, 0)),
        out_shape=jax.ShapeDtypeStruct((N, D), jnp.float32),
    )(eps, h, agg, w1, b1, gm, bm, w2, b2, g, be)


def kernel(x, edge_index, edge_attr, batch, node_enc_W, edge_W, edge_b,
           W1, b1, g_mid, be_mid, W2, b2, eps_arr, gamma, beta):
    h = jnp.take(node_enc_W, x, axis=0)
    row3 = edge_index[0].reshape(NW * NBLK, BCH, CH)
    col3 = edge_index[1].reshape(NW * NBLK, BCH, CH)
    attr8 = jnp.pad(edge_attr, ((0, 0), (0, 1)))

    for l in range(L):
        w8 = jnp.pad(edge_W[l], ((0, 1), (0, 0)))
        ee = _edge_embed(attr8, w8, edge_b[l].reshape(1, D))
        agg = _make_sc_agg(uniform_h=(l == 0))(h, ee, row3, col3)
        h = _mlp(
            eps_arr[l].reshape(1), h, agg,
            W1[l], b1[l].reshape(1, 2 * D),
            g_mid[l].reshape(1, 2 * D), be_mid[l].reshape(1, 2 * D),
            W2[l], b2[l].reshape(1, D),
            gamma[l].reshape(1, D), beta[l].reshape(1, D),
            final=(l == L - 1),
        )
    return h
